# Initial kernel scaffold; baseline (speedup 1.0000x reference)
import jax
import jax.numpy as jnp
from jax.experimental import pallas as pl


def kernel(images, conv1, bn1_s, bn1_b, L1B0_conv1, L1B0_conv2, L1B0_conv3, L1B0_s1, L1B0_b1, L1B0_s2, L1B0_b2, L1B0_s3, L1B0_b3, L1B0_down, L1B0_sd, L1B0_bd, L1B1_conv1, L1B1_conv2, L1B1_conv3, L1B1_s1, L1B1_b1, L1B1_s2, L1B1_b2, L1B1_s3, L1B1_b3, L1B2_conv1, L1B2_conv2, L1B2_conv3, L1B2_s1, L1B2_b1, L1B2_s2, L1B2_b2, L1B2_s3, L1B2_b3, L2B0_conv1, L2B0_conv2, L2B0_conv3, L2B0_s1, L2B0_b1, L2B0_s2, L2B0_b2, L2B0_s3, L2B0_b3, L2B0_down, L2B0_sd, L2B0_bd, L2B1_conv1, L2B1_conv2, L2B1_conv3, L2B1_s1, L2B1_b1, L2B1_s2, L2B1_b2, L2B1_s3, L2B1_b3, L2B2_conv1, L2B2_conv2, L2B2_conv3, L2B2_s1, L2B2_b1, L2B2_s2, L2B2_b2, L2B2_s3, L2B2_b3, L2B3_conv1, L2B3_conv2, L2B3_conv3, L2B3_s1, L2B3_b1, L2B3_s2, L2B3_b2, L2B3_s3, L2B3_b3, L3B0_conv1, L3B0_conv2, L3B0_conv3, L3B0_s1, L3B0_b1, L3B0_s2, L3B0_b2, L3B0_s3, L3B0_b3, L3B0_down, L3B0_sd, L3B0_bd, L3B1_conv1, L3B1_conv2, L3B1_conv3, L3B1_s1, L3B1_b1, L3B1_s2, L3B1_b2, L3B1_s3, L3B1_b3, L3B2_conv1, L3B2_conv2, L3B2_conv3, L3B2_s1, L3B2_b1, L3B2_s2, L3B2_b2, L3B2_s3, L3B2_b3, L3B3_conv1, L3B3_conv2, L3B3_conv3, L3B3_s1, L3B3_b1, L3B3_s2, L3B3_b2, L3B3_s3, L3B3_b3, L3B4_conv1, L3B4_conv2, L3B4_conv3, L3B4_s1, L3B4_b1, L3B4_s2, L3B4_b2, L3B4_s3, L3B4_b3, L3B5_conv1, L3B5_conv2, L3B5_conv3, L3B5_s1, L3B5_b1, L3B5_s2, L3B5_b2, L3B5_s3, L3B5_b3, L4B0_conv1, L4B0_conv2, L4B0_conv3, L4B0_s1, L4B0_b1, L4B0_s2, L4B0_b2, L4B0_s3, L4B0_b3, L4B0_down, L4B0_sd, L4B0_bd, L4B1_conv1, L4B1_conv2, L4B1_conv3, L4B1_s1, L4B1_b1, L4B1_s2, L4B1_b2, L4B1_s3, L4B1_b3, L4B2_conv1, L4B2_conv2, L4B2_conv3, L4B2_s1, L4B2_b1, L4B2_s2, L4B2_b2, L4B2_s3, L4B2_b3, proj_w, proj_s, proj_b):
    raise NotImplementedError("write your pallas kernel here")


# speedup vs baseline: 1.7311x; 1.7311x over previous
import functools

import jax
import jax.numpy as jnp
from jax.experimental import pallas as pl
from jax.experimental.pallas import tpu as pltpu

_TM = 512
_VMEM = 100 * 1024 * 1024


def _cdiv(a, b):
    return (a + b - 1) // b


def _interior_mask(g, geom):
    r = jax.lax.rem(g, geom["img_p"])
    w = jax.lax.rem(r, geom["Wp"])
    return ((r >= geom["Wp"]) & (r < (geom["H"] + 1) * geom["Wp"])
            & (w >= 1) & (w <= geom["W"]))


def _rows_iota(n, base):
    return jax.lax.broadcasted_iota(jnp.int32, (n, 1), 0) + base


def _mm_body(*refs, relu, has_res, geom, tm):
    if has_res:
        x_ref, w_ref, s_ref, b_ref, r_ref, o_ref = refs
    else:
        x_ref, w_ref, s_ref, b_ref, o_ref = refs
    y = jnp.dot(x_ref[...], w_ref[...], preferred_element_type=jnp.float32)
    y = y * s_ref[...] + b_ref[...]
    if has_res:
        y = y + r_ref[...].astype(jnp.float32)
    if relu:
        y = jnp.maximum(y, 0.0)
    if geom is not None:
        g = _rows_iota(y.shape[0], pl.program_id(0) * tm)
        y = jnp.where(_interior_mask(g, geom), y, 0.0)
    o_ref[...] = y.astype(o_ref.dtype)


def _mm(x, w, s, b, relu, res=None, out_dtype=jnp.bfloat16, geom=None):
    M, K = x.shape
    N = w.shape[1]
    tm = min(_TM, M)
    tn = min(N, 512)
    grid = (_cdiv(M, tm), N // tn)
    in_specs = [
        pl.BlockSpec((tm, K), lambda i, j: (i, 0)),
        pl.BlockSpec((K, tn), lambda i, j: (0, j)),
        pl.BlockSpec((1, tn), lambda i, j: (0, j)),
        pl.BlockSpec((1, tn), lambda i, j: (0, j)),
    ]
    args = [x.astype(jnp.bfloat16), w, s, b]
    if res is not None:
        in_specs.append(pl.BlockSpec((tm, tn), lambda i, j: (i, j)))
        args.append(res.astype(jnp.bfloat16))
    return pl.pallas_call(
        functools.partial(_mm_body, relu=relu, has_res=res is not None,
                          geom=geom, tm=tm),
        out_shape=jax.ShapeDtypeStruct((M, N), out_dtype),
        grid=grid,
        in_specs=in_specs,
        out_specs=pl.BlockSpec((tm, tn), lambda i, j: (i, j)),
        compiler_params=pltpu.CompilerParams(
            dimension_semantics=("parallel", "parallel"),
            vmem_limit_bytes=_VMEM),
    )(*args)


def _bneck_body(*refs, geom, has_down):
    if has_down:
        (pv, cu, nx, w1, s1, b1, w2, s2, b2, w3, s3, b3,
         wd, sd, bd, out) = refs
    else:
        pv, cu, nx, w1, s1, b1, w2, s2, b2, w3, s3, b3, out = refs
    tm = cu.shape[0]
    hal = geom["hal"]
    C = w1.shape[1]
    base = pl.program_id(0) * tm

    hw = jnp.concatenate([pv[64 - hal:, :], cu[...], nx[:hal, :]], axis=0)
    t1 = jnp.dot(hw, w1[...], preferred_element_type=jnp.float32)
    t1 = jnp.maximum(t1 * s1[...] + b1[...], 0.0)
    t1 = jnp.where(_interior_mask(_rows_iota(tm + 2 * hal, base - hal), geom),
                   t1, 0.0).astype(jnp.bfloat16)

    acc = None
    for dy in range(3):
        for dx in range(3):
            d = dy * geom["Wp"] + dx - hal
            t = dy * 3 + dx
            p = jnp.dot(t1[hal + d:hal + d + tm, :], w2[t * C:(t + 1) * C, :],
                        preferred_element_type=jnp.float32)
            acc = p if acc is None else acc + p
    ok = _interior_mask(_rows_iota(tm, base), geom)
    t2 = jnp.where(ok, jnp.maximum(acc * s2[...] + b2[...], 0.0),
                   0.0).astype(jnp.bfloat16)

    y = jnp.dot(t2, w3[...], preferred_element_type=jnp.float32)
    y = y * s3[...] + b3[...]
    if has_down:
        idn = jnp.dot(cu[...], wd[...], preferred_element_type=jnp.float32)
        idn = idn * sd[...] + bd[...]
    else:
        idn = cu[...].astype(jnp.float32)
    y = jnp.maximum(y + idn, 0.0)
    out[...] = jnp.where(ok, y, 0.0).astype(out.dtype)


def _bneck(x, geom, w1, s1, b1, w2, s2, b2, w3, s3, b3, down=None):
    M, Cin = x.shape
    C = w1.shape[1]
    C4 = w3.shape[1]
    tm = _TM
    ni = M // tm
    nh = M // 64
    full = lambda a: pl.BlockSpec(a.shape, lambda i: (0, 0))
    in_specs = [
        pl.BlockSpec((64, Cin), lambda i: (jnp.maximum(i * 8 - 1, 0), 0)),
        pl.BlockSpec((tm, Cin), lambda i: (i, 0)),
        pl.BlockSpec((64, Cin), lambda i: (jnp.minimum(i * 8 + 8, nh - 1), 0)),
    ]
    args = [x, x, x, w1, s1, b1, w2, s2, b2, w3, s3, b3]
    in_specs += [full(a) for a in args[3:]]
    if down is not None:
        wd, sd, bd = down
        args += [wd, sd, bd]
        in_specs += [full(wd), full(sd), full(bd)]
    return pl.pallas_call(
        functools.partial(_bneck_body, geom=geom, has_down=down is not None),
        out_shape=jax.ShapeDtypeStruct((M, C4), jnp.bfloat16),
        grid=(ni,),
        in_specs=in_specs,
        out_specs=pl.BlockSpec((tm, C4), lambda i: (i, 0)),
        compiler_params=pltpu.CompilerParams(
            dimension_semantics=("parallel",),
            vmem_limit_bytes=_VMEM),
    )(*args)


def _pool_body(*refs):
    acc = refs[0][...]
    for r in refs[1:-1]:
        acc = jnp.maximum(acc, r[...])
    refs[-1][...] = acc


def _gap_proj_body(x_ref, w_ref, s_ref, b_ref, o_ref, *, hw):
    f = jnp.sum(x_ref[...].astype(jnp.float32), axis=1) * (1.0 / hw)
    y = jnp.dot(f.astype(jnp.bfloat16), w_ref[...],
                preferred_element_type=jnp.float32)
    o_ref[...] = y * s_ref[...] + b_ref[...]


def _geom(H, W):
    Hp, Wp = H + 2, W + 2
    img = Hp * Wp
    img_p = _cdiv(img, 16) * 16
    return {"H": H, "W": W, "Wp": Wp, "img": img, "img_p": img_p,
            "hal": Wp + 1}


def _to_layout(x, geom):
    B, H, W, C = x.shape
    xp = jnp.pad(x, ((0, 0), (1, 1), (1, 1), (0, 0)))
    xp = xp.reshape(B, geom["img"], C)
    xp = jnp.pad(xp, ((0, 0), (0, geom["img_p"] - geom["img"]), (0, 0)))
    return xp.reshape(B * geom["img_p"], C)


def _from_layout(x, geom, B):
    C = x.shape[1]
    return (x.reshape(B, geom["img_p"], C)[:, :geom["img"], :]
            .reshape(B, geom["H"] + 2, geom["Wp"], C))


def _im2col_s2(xpad, Ho, Wo, k=3):
    cols = [xpad[:, dy:dy + 2 * Ho - 1:2, dx:dx + 2 * Wo - 1:2, :]
            for dy in range(k) for dx in range(k)]
    B = xpad.shape[0]
    return jnp.concatenate(cols, axis=-1).reshape(B * Ho * Wo, -1)


def _block_s2(x, gin, gout, B, p):
    (w1, s1, b1, w2, s2, b2, w3, s3, b3, wd, sd, bd) = p
    Ho, Wo = gout["H"], gout["W"]
    t1 = _mm(x, w1, s1, b1, relu=True, geom=gin)
    t1p = _from_layout(t1, gin, B)
    rows = _im2col_s2(t1p, Ho, Wo)
    t2 = _mm(rows, w2, s2, b2, relu=True)
    xc = _from_layout(x, gin, B)[:, 1:2 * Ho:2, 1:2 * Wo:2, :]
    idn = _mm(xc.reshape(B * Ho * Wo, -1), wd, sd, bd, relu=False)
    y = _mm(t2, w3, s3, b3, relu=True, res=idn)
    return _to_layout(y.reshape(B, Ho, Wo, -1), gout)


def kernel(images, conv1, bn1_s, bn1_b, L1B0_conv1, L1B0_conv2, L1B0_conv3, L1B0_s1, L1B0_b1, L1B0_s2, L1B0_b2, L1B0_s3, L1B0_b3, L1B0_down, L1B0_sd, L1B0_bd, L1B1_conv1, L1B1_conv2, L1B1_conv3, L1B1_s1, L1B1_b1, L1B1_s2, L1B1_b2, L1B1_s3, L1B1_b3, L1B2_conv1, L1B2_conv2, L1B2_conv3, L1B2_s1, L1B2_b1, L1B2_s2, L1B2_b2, L1B2_s3, L1B2_b3, L2B0_conv1, L2B0_conv2, L2B0_conv3, L2B0_s1, L2B0_b1, L2B0_s2, L2B0_b2, L2B0_s3, L2B0_b3, L2B0_down, L2B0_sd, L2B0_bd, L2B1_conv1, L2B1_conv2, L2B1_conv3, L2B1_s1, L2B1_b1, L2B1_s2, L2B1_b2, L2B1_s3, L2B1_b3, L2B2_conv1, L2B2_conv2, L2B2_conv3, L2B2_s1, L2B2_b1, L2B2_s2, L2B2_b2, L2B2_s3, L2B2_b3, L2B3_conv1, L2B3_conv2, L2B3_conv3, L2B3_s1, L2B3_b1, L2B3_s2, L2B3_b2, L2B3_s3, L2B3_b3, L3B0_conv1, L3B0_conv2, L3B0_conv3, L3B0_s1, L3B0_b1, L3B0_s2, L3B0_b2, L3B0_s3, L3B0_b3, L3B0_down, L3B0_sd, L3B0_bd, L3B1_conv1, L3B1_conv2, L3B1_conv3, L3B1_s1, L3B1_b1, L3B1_s2, L3B1_b2, L3B1_s3, L3B1_b3, L3B2_conv1, L3B2_conv2, L3B2_conv3, L3B2_s1, L3B2_b1, L3B2_s2, L3B2_b2, L3B2_s3, L3B2_b3, L3B3_conv1, L3B3_conv2, L3B3_conv3, L3B3_s1, L3B3_b1, L3B3_s2, L3B3_b2, L3B3_s3, L3B3_b3, L3B4_conv1, L3B4_conv2, L3B4_conv3, L3B4_s1, L3B4_b1, L3B4_s2, L3B4_b2, L3B4_s3, L3B4_b3, L3B5_conv1, L3B5_conv2, L3B5_conv3, L3B5_s1, L3B5_b1, L3B5_s2, L3B5_b2, L3B5_s3, L3B5_b3, L4B0_conv1, L4B0_conv2, L4B0_conv3, L4B0_s1, L4B0_b1, L4B0_s2, L4B0_b2, L4B0_s3, L4B0_b3, L4B0_down, L4B0_sd, L4B0_bd, L4B1_conv1, L4B1_conv2, L4B1_conv3, L4B1_s1, L4B1_b1, L4B1_s2, L4B1_b2, L4B1_s3, L4B1_b3, L4B2_conv1, L4B2_conv2, L4B2_conv3, L4B2_s1, L4B2_b1, L4B2_s2, L4B2_b2, L4B2_s3, L4B2_b3, proj_w, proj_s, proj_b):
    B = images.shape[0]
    g1, g2, g3, g4 = _geom(56, 56), _geom(28, 28), _geom(14, 14), _geom(7, 7)

    x = jnp.transpose(images, (0, 2, 3, 1)).astype(jnp.bfloat16)
    xp = jnp.pad(x, ((0, 0), (3, 3), (3, 3), (0, 0)))
    cols = [xp[:, dy:dy + 223:2, dx:dx + 223:2, :]
            for dy in range(7) for dx in range(7)]
    cols.append(jnp.zeros((B, 112, 112, 256 - 147), jnp.bfloat16))
    rows = jnp.concatenate(cols, axis=-1).reshape(B * 112 * 112, 256)
    y = _mm(rows, conv1, bn1_s, bn1_b, relu=True).reshape(B, 112, 112, 128)

    yp = jnp.pad(y, ((0, 0), (1, 1), (1, 1), (0, 0)),
                 constant_values=-jnp.inf)
    taps = [yp[:, dy:dy + 111:2, dx:dx + 111:2, :].reshape(B * 56 * 56, 128)
            for dy in range(3) for dx in range(3)]
    M1 = B * 56 * 56
    pooled = pl.pallas_call(
        _pool_body,
        out_shape=jax.ShapeDtypeStruct((M1, 128), jnp.bfloat16),
        grid=(M1 // _TM,),
        in_specs=[pl.BlockSpec((_TM, 128), lambda i: (i, 0))] * 9,
        out_specs=pl.BlockSpec((_TM, 128), lambda i: (i, 0)),
        compiler_params=pltpu.CompilerParams(
            dimension_semantics=("parallel",), vmem_limit_bytes=_VMEM),
    )(*taps)
    x = _to_layout(pooled.reshape(B, 56, 56, 128), g1)

    x = _bneck(x, g1, L1B0_conv1, L1B0_s1, L1B0_b1, L1B0_conv2, L1B0_s2,
               L1B0_b2, L1B0_conv3, L1B0_s3, L1B0_b3,
               down=(L1B0_down, L1B0_sd, L1B0_bd))
    x = _bneck(x, g1, L1B1_conv1, L1B1_s1, L1B1_b1, L1B1_conv2, L1B1_s2,
               L1B1_b2, L1B1_conv3, L1B1_s3, L1B1_b3)
    x = _bneck(x, g1, L1B2_conv1, L1B2_s1, L1B2_b1, L1B2_conv2, L1B2_s2,
               L1B2_b2, L1B2_conv3, L1B2_s3, L1B2_b3)

    x = _block_s2(x, g1, g2, B, (L2B0_conv1, L2B0_s1, L2B0_b1, L2B0_conv2,
                                 L2B0_s2, L2B0_b2, L2B0_conv3, L2B0_s3,
                                 L2B0_b3, L2B0_down, L2B0_sd, L2B0_bd))
    x = _bneck(x, g2, L2B1_conv1, L2B1_s1, L2B1_b1, L2B1_conv2, L2B1_s2,
               L2B1_b2, L2B1_conv3, L2B1_s3, L2B1_b3)
    x = _bneck(x, g2, L2B2_conv1, L2B2_s1, L2B2_b1, L2B2_conv2, L2B2_s2,
               L2B2_b2, L2B2_conv3, L2B2_s3, L2B2_b3)
    x = _bneck(x, g2, L2B3_conv1, L2B3_s1, L2B3_b1, L2B3_conv2, L2B3_s2,
               L2B3_b2, L2B3_conv3, L2B3_s3, L2B3_b3)

    x = _block_s2(x, g2, g3, B, (L3B0_conv1, L3B0_s1, L3B0_b1, L3B0_conv2,
                                 L3B0_s2, L3B0_b2, L3B0_conv3, L3B0_s3,
                                 L3B0_b3, L3B0_down, L3B0_sd, L3B0_bd))
    for p in ((L3B1_conv1, L3B1_s1, L3B1_b1, L3B1_conv2, L3B1_s2, L3B1_b2,
               L3B1_conv3, L3B1_s3, L3B1_b3),
              (L3B2_conv1, L3B2_s1, L3B2_b1, L3B2_conv2, L3B2_s2, L3B2_b2,
               L3B2_conv3, L3B2_s3, L3B2_b3),
              (L3B3_conv1, L3B3_s1, L3B3_b1, L3B3_conv2, L3B3_s2, L3B3_b2,
               L3B3_conv3, L3B3_s3, L3B3_b3),
              (L3B4_conv1, L3B4_s1, L3B4_b1, L3B4_conv2, L3B4_s2, L3B4_b2,
               L3B4_conv3, L3B4_s3, L3B4_b3),
              (L3B5_conv1, L3B5_s1, L3B5_b1, L3B5_conv2, L3B5_s2, L3B5_b2,
               L3B5_conv3, L3B5_s3, L3B5_b3)):
        x = _bneck(x, g3, *p)

    x = _block_s2(x, g3, g4, B, (L4B0_conv1, L4B0_s1, L4B0_b1, L4B0_conv2,
                                 L4B0_s2, L4B0_b2, L4B0_conv3, L4B0_s3,
                                 L4B0_b3, L4B0_down, L4B0_sd, L4B0_bd))
    x = _bneck(x, g4, L4B1_conv1, L4B1_s1, L4B1_b1, L4B1_conv2, L4B1_s2,
               L4B1_b2, L4B1_conv3, L4B1_s3, L4B1_b3)
    x = _bneck(x, g4, L4B2_conv1, L4B2_s1, L4B2_b1, L4B2_conv2, L4B2_s2,
               L4B2_b2, L4B2_conv3, L4B2_s3, L4B2_b3)

    x3 = x.reshape(B, g4["img_p"], 2048)
    out = pl.pallas_call(
        functools.partial(_gap_proj_body, hw=49.0),
        out_shape=jax.ShapeDtypeStruct((B, 512), jnp.float32),
        compiler_params=pltpu.CompilerParams(vmem_limit_bytes=_VMEM),
    )(x3, proj_w, proj_s, proj_b)
    return out.reshape(B, 1, 512)


# speedup vs baseline: 2.1692x; 1.2531x over previous
import functools

import jax
import jax.numpy as jnp
from jax.experimental import pallas as pl
from jax.experimental.pallas import tpu as pltpu

_TM = 512
_VMEM = 100 * 1024 * 1024


def _cdiv(a, b):
    return (a + b - 1) // b


def _interior_mask(g, geom):
    r = jax.lax.rem(g, geom["img_p"])
    w = jax.lax.rem(r, geom["Wp"])
    return ((r >= geom["Wp"]) & (r < (geom["H"] + 1) * geom["Wp"])
            & (w >= 1) & (w <= geom["W"]))


def _rows_iota(n, base):
    return jax.lax.broadcasted_iota(jnp.int32, (n, 1), 0) + base


def _mm_body(*refs, relu, has_res, geom, tm):
    if has_res:
        x_ref, w_ref, s_ref, b_ref, r_ref, o_ref = refs
    else:
        x_ref, w_ref, s_ref, b_ref, o_ref = refs
    y = jnp.dot(x_ref[...], w_ref[...], preferred_element_type=jnp.float32)
    y = y * s_ref[...] + b_ref[...]
    if has_res:
        y = y + r_ref[...].astype(jnp.float32)
    if relu:
        y = jnp.maximum(y, 0.0)
    if geom is not None:
        g = _rows_iota(y.shape[0], pl.program_id(0) * tm)
        y = jnp.where(_interior_mask(g, geom), y, 0.0)
    o_ref[...] = y.astype(o_ref.dtype)


def _mm(x, w, s, b, relu, res=None, out_dtype=jnp.bfloat16, geom=None):
    M, K = x.shape
    N = w.shape[1]
    tm = min(_TM, M)
    tn = min(N, 512)
    grid = (_cdiv(M, tm), N // tn)
    in_specs = [
        pl.BlockSpec((tm, K), lambda i, j: (i, 0)),
        pl.BlockSpec((K, tn), lambda i, j: (0, j)),
        pl.BlockSpec((1, tn), lambda i, j: (0, j)),
        pl.BlockSpec((1, tn), lambda i, j: (0, j)),
    ]
    args = [x.astype(jnp.bfloat16), w, s, b]
    if res is not None:
        in_specs.append(pl.BlockSpec((tm, tn), lambda i, j: (i, j)))
        args.append(res.astype(jnp.bfloat16))
    return pl.pallas_call(
        functools.partial(_mm_body, relu=relu, has_res=res is not None,
                          geom=geom, tm=tm),
        out_shape=jax.ShapeDtypeStruct((M, N), out_dtype),
        grid=grid,
        in_specs=in_specs,
        out_specs=pl.BlockSpec((tm, tn), lambda i, j: (i, j)),
        compiler_params=pltpu.CompilerParams(
            dimension_semantics=("parallel", "parallel"),
            vmem_limit_bytes=_VMEM),
    )(*args)


def _bneck_body(*refs, geom, has_down):
    if has_down:
        (pv, cu, nx, w1, s1, b1, w2, s2, b2, w3, s3, b3,
         wd, sd, bd, out) = refs
    else:
        pv, cu, nx, w1, s1, b1, w2, s2, b2, w3, s3, b3, out = refs
    tm = cu.shape[0]
    hal = geom["hal"]
    C = w1.shape[1]
    base = pl.program_id(0) * tm

    hw = jnp.concatenate([pv[64 - hal:, :], cu[...], nx[:hal, :]], axis=0)
    t1 = jnp.dot(hw, w1[...], preferred_element_type=jnp.float32)
    t1 = jnp.maximum(t1 * s1[...] + b1[...], 0.0)
    t1 = jnp.where(_interior_mask(_rows_iota(tm + 2 * hal, base - hal), geom),
                   t1, 0.0).astype(jnp.bfloat16)

    acc = None
    for dy in range(3):
        for dx in range(3):
            d = dy * geom["Wp"] + dx - hal
            t = dy * 3 + dx
            p = jnp.dot(t1[hal + d:hal + d + tm, :], w2[t * C:(t + 1) * C, :],
                        preferred_element_type=jnp.float32)
            acc = p if acc is None else acc + p
    ok = _interior_mask(_rows_iota(tm, base), geom)
    t2 = jnp.where(ok, jnp.maximum(acc * s2[...] + b2[...], 0.0),
                   0.0).astype(jnp.bfloat16)

    y = jnp.dot(t2, w3[...], preferred_element_type=jnp.float32)
    y = y * s3[...] + b3[...]
    if has_down:
        idn = jnp.dot(cu[...], wd[...], preferred_element_type=jnp.float32)
        idn = idn * sd[...] + bd[...]
    else:
        idn = cu[...].astype(jnp.float32)
    y = jnp.maximum(y + idn, 0.0)
    out[...] = jnp.where(ok, y, 0.0).astype(out.dtype)


def _bneck(x, geom, w1, s1, b1, w2, s2, b2, w3, s3, b3, down=None):
    M, Cin = x.shape
    C = w1.shape[1]
    C4 = w3.shape[1]
    tm = _TM
    ni = M // tm
    nh = M // 64
    full = lambda a: pl.BlockSpec(a.shape, lambda i: (0, 0))
    in_specs = [
        pl.BlockSpec((64, Cin), lambda i: (jnp.maximum(i * 8 - 1, 0), 0)),
        pl.BlockSpec((tm, Cin), lambda i: (i, 0)),
        pl.BlockSpec((64, Cin), lambda i: (jnp.minimum(i * 8 + 8, nh - 1), 0)),
    ]
    args = [x, x, x, w1, s1, b1, w2, s2, b2, w3, s3, b3]
    in_specs += [full(a) for a in args[3:]]
    if down is not None:
        wd, sd, bd = down
        args += [wd, sd, bd]
        in_specs += [full(wd), full(sd), full(bd)]
    return pl.pallas_call(
        functools.partial(_bneck_body, geom=geom, has_down=down is not None),
        out_shape=jax.ShapeDtypeStruct((M, C4), jnp.bfloat16),
        grid=(ni,),
        in_specs=in_specs,
        out_specs=pl.BlockSpec((tm, C4), lambda i: (i, 0)),
        compiler_params=pltpu.CompilerParams(
            dimension_semantics=("parallel",),
            vmem_limit_bytes=_VMEM),
    )(*args)


def _pool_body(*refs):
    acc = refs[0][...]
    for r in refs[1:-1]:
        acc = jnp.maximum(acc, r[...])
    refs[-1][...] = acc


def _stem_body(pv, cu, nx, w_ref, s_ref, b_ref, o_ref, *, tm, W):
    hw = jnp.concatenate([pv[7:, :], cu[...], nx[:1, :]], axis=0)
    y = jnp.dot(hw, w_ref[...], preferred_element_type=jnp.float32)
    y = jnp.maximum(y * s_ref[...] + b_ref[...], 0.0)
    wcol = jax.lax.rem(_rows_iota(tm, pl.program_id(0) * tm), W)
    left = jnp.where(wcol >= 1, y[0:tm, :], -jnp.inf)
    right = jnp.where(wcol <= W - 2, y[2:tm + 2, :], -jnp.inf)
    o_ref[...] = jnp.maximum(jnp.maximum(y[1:tm + 1, :], left),
                             right).astype(o_ref.dtype)


def _gap_proj_body(x_ref, w_ref, s_ref, b_ref, o_ref, *, hw):
    f = jnp.sum(x_ref[...].astype(jnp.float32), axis=1) * (1.0 / hw)
    y = jnp.dot(f.astype(jnp.bfloat16), w_ref[...],
                preferred_element_type=jnp.float32)
    o_ref[...] = y * s_ref[...] + b_ref[...]


def _geom(H, W):
    Hp, Wp = H + 2, W + 2
    img = Hp * Wp
    img_p = _cdiv(img, 16) * 16
    return {"H": H, "W": W, "Wp": Wp, "img": img, "img_p": img_p,
            "hal": Wp + 1}


def _to_layout(x, geom):
    B, H, W, C = x.shape
    xp = jnp.pad(x, ((0, 0), (1, 1), (1, 1), (0, 0)))
    xp = xp.reshape(B, geom["img"], C)
    xp = jnp.pad(xp, ((0, 0), (0, geom["img_p"] - geom["img"]), (0, 0)))
    return xp.reshape(B * geom["img_p"], C)


def _from_layout(x, geom, B):
    C = x.shape[1]
    return (x.reshape(B, geom["img_p"], C)[:, :geom["img"], :]
            .reshape(B, geom["H"] + 2, geom["Wp"], C))


def _im2col_s2(xpad, Ho, Wo, k=3):
    cols = [xpad[:, dy:dy + 2 * Ho - 1:2, dx:dx + 2 * Wo - 1:2, :]
            for dy in range(k) for dx in range(k)]
    B = xpad.shape[0]
    return jnp.concatenate(cols, axis=-1).reshape(B * Ho * Wo, -1)


def _block_s2(x, gin, gout, B, p):
    (w1, s1, b1, w2, s2, b2, w3, s3, b3, wd, sd, bd) = p
    Ho, Wo = gout["H"], gout["W"]
    t1 = _mm(x, w1, s1, b1, relu=True, geom=gin)
    t1p = _from_layout(t1, gin, B)
    rows = _im2col_s2(t1p, Ho, Wo)
    t2 = _mm(rows, w2, s2, b2, relu=True)
    xc = _from_layout(x, gin, B)[:, 1:2 * Ho:2, 1:2 * Wo:2, :]
    idn = _mm(xc.reshape(B * Ho * Wo, -1), wd, sd, bd, relu=False)
    y = _mm(t2, w3, s3, b3, relu=True, res=idn)
    return _to_layout(y.reshape(B, Ho, Wo, -1), gout)


def kernel(images, conv1, bn1_s, bn1_b, L1B0_conv1, L1B0_conv2, L1B0_conv3, L1B0_s1, L1B0_b1, L1B0_s2, L1B0_b2, L1B0_s3, L1B0_b3, L1B0_down, L1B0_sd, L1B0_bd, L1B1_conv1, L1B1_conv2, L1B1_conv3, L1B1_s1, L1B1_b1, L1B1_s2, L1B1_b2, L1B1_s3, L1B1_b3, L1B2_conv1, L1B2_conv2, L1B2_conv3, L1B2_s1, L1B2_b1, L1B2_s2, L1B2_b2, L1B2_s3, L1B2_b3, L2B0_conv1, L2B0_conv2, L2B0_conv3, L2B0_s1, L2B0_b1, L2B0_s2, L2B0_b2, L2B0_s3, L2B0_b3, L2B0_down, L2B0_sd, L2B0_bd, L2B1_conv1, L2B1_conv2, L2B1_conv3, L2B1_s1, L2B1_b1, L2B1_s2, L2B1_b2, L2B1_s3, L2B1_b3, L2B2_conv1, L2B2_conv2, L2B2_conv3, L2B2_s1, L2B2_b1, L2B2_s2, L2B2_b2, L2B2_s3, L2B2_b3, L2B3_conv1, L2B3_conv2, L2B3_conv3, L2B3_s1, L2B3_b1, L2B3_s2, L2B3_b2, L2B3_s3, L2B3_b3, L3B0_conv1, L3B0_conv2, L3B0_conv3, L3B0_s1, L3B0_b1, L3B0_s2, L3B0_b2, L3B0_s3, L3B0_b3, L3B0_down, L3B0_sd, L3B0_bd, L3B1_conv1, L3B1_conv2, L3B1_conv3, L3B1_s1, L3B1_b1, L3B1_s2, L3B1_b2, L3B1_s3, L3B1_b3, L3B2_conv1, L3B2_conv2, L3B2_conv3, L3B2_s1, L3B2_b1, L3B2_s2, L3B2_b2, L3B2_s3, L3B2_b3, L3B3_conv1, L3B3_conv2, L3B3_conv3, L3B3_s1, L3B3_b1, L3B3_s2, L3B3_b2, L3B3_s3, L3B3_b3, L3B4_conv1, L3B4_conv2, L3B4_conv3, L3B4_s1, L3B4_b1, L3B4_s2, L3B4_b2, L3B4_s3, L3B4_b3, L3B5_conv1, L3B5_conv2, L3B5_conv3, L3B5_s1, L3B5_b1, L3B5_s2, L3B5_b2, L3B5_s3, L3B5_b3, L4B0_conv1, L4B0_conv2, L4B0_conv3, L4B0_s1, L4B0_b1, L4B0_s2, L4B0_b2, L4B0_s3, L4B0_b3, L4B0_down, L4B0_sd, L4B0_bd, L4B1_conv1, L4B1_conv2, L4B1_conv3, L4B1_s1, L4B1_b1, L4B1_s2, L4B1_b2, L4B1_s3, L4B1_b3, L4B2_conv1, L4B2_conv2, L4B2_conv3, L4B2_s1, L4B2_b1, L4B2_s2, L4B2_b2, L4B2_s3, L4B2_b3, proj_w, proj_s, proj_b):
    B = images.shape[0]
    g1, g2, g3, g4 = _geom(56, 56), _geom(28, 28), _geom(14, 14), _geom(7, 7)

    x = jnp.transpose(images, (0, 2, 3, 1)).astype(jnp.bfloat16)
    xp = jnp.pad(x, ((0, 0), (3, 3), (3, 3), (0, 0)))
    cols = [xp[:, dy:dy + 223:2, dx:dx + 223:2, :]
            for dy in range(7) for dx in range(7)]
    cols.append(jnp.zeros((B, 112, 112, 160 - 147), jnp.bfloat16))
    rows = jnp.concatenate(cols, axis=-1).reshape(B * 112 * 112, 160)
    M0 = B * 112 * 112
    nh0 = M0 // 8
    wmax = pl.pallas_call(
        functools.partial(_stem_body, tm=_TM, W=112),
        out_shape=jax.ShapeDtypeStruct((M0, 128), jnp.bfloat16),
        grid=(M0 // _TM,),
        in_specs=[
            pl.BlockSpec((8, 160), lambda i: (jnp.maximum(i * 64 - 1, 0), 0)),
            pl.BlockSpec((_TM, 160), lambda i: (i, 0)),
            pl.BlockSpec((8, 160), lambda i: (jnp.minimum(i * 64 + 64,
                                                          nh0 - 1), 0)),
            pl.BlockSpec((160, 128), lambda i: (0, 0)),
            pl.BlockSpec((1, 128), lambda i: (0, 0)),
            pl.BlockSpec((1, 128), lambda i: (0, 0)),
        ],
        out_specs=pl.BlockSpec((_TM, 128), lambda i: (i, 0)),
        compiler_params=pltpu.CompilerParams(
            dimension_semantics=("parallel",), vmem_limit_bytes=_VMEM),
    )(rows, rows, rows, conv1[:160, :], bn1_s, bn1_b)

    yp = jnp.pad(wmax.reshape(B, 112, 112, 128), ((0, 0), (1, 1), (0, 0),
                                                  (0, 0)),
                 constant_values=-jnp.inf)
    taps = [yp[:, dy:dy + 111:2, 0:112:2, :].reshape(B * 56 * 56, 128)
            for dy in range(3)]
    M1 = B * 56 * 56
    pooled = pl.pallas_call(
        _pool_body,
        out_shape=jax.ShapeDtypeStruct((M1, 128), jnp.bfloat16),
        grid=(M1 // _TM,),
        in_specs=[pl.BlockSpec((_TM, 128), lambda i: (i, 0))] * 3,
        out_specs=pl.BlockSpec((_TM, 128), lambda i: (i, 0)),
        compiler_params=pltpu.CompilerParams(
            dimension_semantics=("parallel",), vmem_limit_bytes=_VMEM),
    )(*taps)
    x = _to_layout(pooled.reshape(B, 56, 56, 128), g1)

    x = _bneck(x, g1, L1B0_conv1, L1B0_s1, L1B0_b1, L1B0_conv2, L1B0_s2,
               L1B0_b2, L1B0_conv3, L1B0_s3, L1B0_b3,
               down=(L1B0_down, L1B0_sd, L1B0_bd))
    x = _bneck(x, g1, L1B1_conv1, L1B1_s1, L1B1_b1, L1B1_conv2, L1B1_s2,
               L1B1_b2, L1B1_conv3, L1B1_s3, L1B1_b3)
    x = _bneck(x, g1, L1B2_conv1, L1B2_s1, L1B2_b1, L1B2_conv2, L1B2_s2,
               L1B2_b2, L1B2_conv3, L1B2_s3, L1B2_b3)

    x = _block_s2(x, g1, g2, B, (L2B0_conv1, L2B0_s1, L2B0_b1, L2B0_conv2,
                                 L2B0_s2, L2B0_b2, L2B0_conv3, L2B0_s3,
                                 L2B0_b3, L2B0_down, L2B0_sd, L2B0_bd))
    x = _bneck(x, g2, L2B1_conv1, L2B1_s1, L2B1_b1, L2B1_conv2, L2B1_s2,
               L2B1_b2, L2B1_conv3, L2B1_s3, L2B1_b3)
    x = _bneck(x, g2, L2B2_conv1, L2B2_s1, L2B2_b1, L2B2_conv2, L2B2_s2,
               L2B2_b2, L2B2_conv3, L2B2_s3, L2B2_b3)
    x = _bneck(x, g2, L2B3_conv1, L2B3_s1, L2B3_b1, L2B3_conv2, L2B3_s2,
               L2B3_b2, L2B3_conv3, L2B3_s3, L2B3_b3)

    x = _block_s2(x, g2, g3, B, (L3B0_conv1, L3B0_s1, L3B0_b1, L3B0_conv2,
                                 L3B0_s2, L3B0_b2, L3B0_conv3, L3B0_s3,
                                 L3B0_b3, L3B0_down, L3B0_sd, L3B0_bd))
    for p in ((L3B1_conv1, L3B1_s1, L3B1_b1, L3B1_conv2, L3B1_s2, L3B1_b2,
               L3B1_conv3, L3B1_s3, L3B1_b3),
              (L3B2_conv1, L3B2_s1, L3B2_b1, L3B2_conv2, L3B2_s2, L3B2_b2,
               L3B2_conv3, L3B2_s3, L3B2_b3),
              (L3B3_conv1, L3B3_s1, L3B3_b1, L3B3_conv2, L3B3_s2, L3B3_b2,
               L3B3_conv3, L3B3_s3, L3B3_b3),
              (L3B4_conv1, L3B4_s1, L3B4_b1, L3B4_conv2, L3B4_s2, L3B4_b2,
               L3B4_conv3, L3B4_s3, L3B4_b3),
              (L3B5_conv1, L3B5_s1, L3B5_b1, L3B5_conv2, L3B5_s2, L3B5_b2,
               L3B5_conv3, L3B5_s3, L3B5_b3)):
        x = _bneck(x, g3, *p)

    x = _block_s2(x, g3, g4, B, (L4B0_conv1, L4B0_s1, L4B0_b1, L4B0_conv2,
                                 L4B0_s2, L4B0_b2, L4B0_conv3, L4B0_s3,
                                 L4B0_b3, L4B0_down, L4B0_sd, L4B0_bd))
    x = _bneck(x, g4, L4B1_conv1, L4B1_s1, L4B1_b1, L4B1_conv2, L4B1_s2,
               L4B1_b2, L4B1_conv3, L4B1_s3, L4B1_b3)
    x = _bneck(x, g4, L4B2_conv1, L4B2_s1, L4B2_b1, L4B2_conv2, L4B2_s2,
               L4B2_b2, L4B2_conv3, L4B2_s3, L4B2_b3)

    x3 = x.reshape(B, g4["img_p"], 2048)
    out = pl.pallas_call(
        functools.partial(_gap_proj_body, hw=49.0),
        out_shape=jax.ShapeDtypeStruct((B, 512), jnp.float32),
        compiler_params=pltpu.CompilerParams(vmem_limit_bytes=_VMEM),
    )(x3, proj_w, proj_s, proj_b)
    return out.reshape(B, 1, 512)
